# baseline (speedup 1.0000x reference)
import math

import jax
import jax.numpy as jnp
from jax import lax
from jax.experimental import pallas as pl
from jax.experimental.pallas import tpu as pltpu
from jax.experimental.pallas import tpu_sc as plsc

_N = 10000
_E = 320000
_D = 128
_NP = 10112
_NDR = 80
_NC = 2
_NS = 16
_NW = _NC * _NS
_EPW = _E // _NW
_C = 80
_NCHUNK = _EPW // _C
_C1 = 128
_NCH1 = _EPW // _C1
_TAIL1 = _EPW - _NCH1 * _C1
_STRIPE = _NP // _NS
_INV_SCALE = 1.0 / math.sqrt(_D)

_mesh = dict(core_axis_name="c", subcore_axis_name="s", num_cores=_NC,
             num_subcores=_NS)


def _pass1_body(k_hbm, q_hbm, src_hbm, dst_hbm, a_out, mx_out,
                srcall, dstall, kr0, qr0, kr1, qr1, abuf, mbuf,
                sem0, sem1):
  cid = lax.axis_index("c")
  sid = lax.axis_index("s")
  wid = sid * _NC + cid
  ebase = wid * _EPW
  mbuf[...] = jnp.full((16,), -jnp.inf, jnp.float32)
  pltpu.sync_copy(src_hbm.at[pl.ds(ebase, _EPW)], srcall)
  pltpu.sync_copy(dst_hbm.at[pl.ds(ebase, _EPW)], dstall)
  iota = lax.iota(jnp.int32, 16)

  def start(i, kr, qr, sem, n=_C1):
    pltpu.async_copy(k_hbm.at[srcall.at[pl.ds(i * _C1, n)]],
                     kr.at[pl.ds(0, n)], sem)
    pltpu.async_copy(q_hbm.at[dstall.at[pl.ds(i * _C1, n)]],
                     qr.at[pl.ds(0, n)], sem)

  def wait(kr, qr, sem, n=_C1):
    pltpu.make_async_copy(k_hbm.at[srcall.at[pl.ds(0, n)]],
                          kr.at[pl.ds(0, n)], sem).wait()
    pltpu.make_async_copy(q_hbm.at[dstall.at[pl.ds(0, n)]],
                          qr.at[pl.ds(0, n)], sem).wait()

  def compute(i, kr, qr, n=_C1):
    @pl.loop(0, n // 16)
    def _grp(g):
      res = jnp.zeros((16,), jnp.float32)
      for l in range(16):
        e = g * 16 + l
        acc = kr[e, pl.ds(0, 16)] * qr[e, pl.ds(0, 16)]
        for j in range(1, 8):
          acc = acc + kr[e, pl.ds(16 * j, 16)] * qr[e, pl.ds(16 * j, 16)]
        for sh in (1, 2, 4, 8):
          acc = acc + acc[jnp.bitwise_xor(iota, sh)]
        res = jnp.where(iota == l, acc, res)
      abuf[pl.ds(i * _C1 + g * 16, 16)] = res * _INV_SCALE

  start(0, kr0, qr0, sem0)

  @pl.loop(0, _NCH1 - 1, step=2)
  def _body(i):
    start(i + 1, kr1, qr1, sem1)
    wait(kr0, qr0, sem0)
    compute(i, kr0, qr0)

    @pl.when(i + 2 < _NCH1)
    def _():
      start(i + 2, kr0, qr0, sem0)

    wait(kr1, qr1, sem1)
    compute(i + 1, kr1, qr1)

  start(_NCH1, kr0, qr0, sem0, n=_TAIL1)
  wait(kr0, qr0, sem0, n=_TAIL1)
  compute(_NCH1, kr0, qr0, n=_TAIL1)

  @pl.loop(0, _EPW // 16)
  def _mx(t):
    mbuf[...] = jnp.maximum(mbuf[...], abuf[pl.ds(t * 16, 16)])

  pltpu.sync_copy(abuf, a_out.at[pl.ds(ebase, _EPW)])
  pltpu.sync_copy(mbuf, mx_out.at[wid])


@jax.jit
def _pass1(k, q, src, dst):
  f = pl.kernel(
      _pass1_body,
      out_type=(jax.ShapeDtypeStruct((_E,), jnp.float32),
                jax.ShapeDtypeStruct((_NW, 16), jnp.float32)),
      mesh=plsc.VectorSubcoreMesh(**_mesh),
      scratch_types=(
          pltpu.VMEM((_EPW,), jnp.int32),
          pltpu.VMEM((_EPW,), jnp.int32),
          pltpu.VMEM((_C1, _D), jnp.float32),
          pltpu.VMEM((_C1, _D), jnp.float32),
          pltpu.VMEM((_C1, _D), jnp.float32),
          pltpu.VMEM((_C1, _D), jnp.float32),
          pltpu.VMEM((_EPW,), jnp.float32),
          pltpu.VMEM((16,), jnp.float32),
          pltpu.SemaphoreType.DMA,
          pltpu.SemaphoreType.DMA,
      ),
  )
  return f(k, q, src, dst)


def _pass2_body(v_hbm, a_hbm, src_hbm, dst_hbm, mx_hbm, s_out, den_out,
                srcv0, srcv1, dstv0, av0, didx0, dstv1, av1, didx1,
                vb0, vb1, oh, mxv, s_shared, den_shared,
                semi0, semi1, semg0, semg1, semv0, semv1, semo):
  cid = lax.axis_index("c")
  sid = lax.axis_index("s")
  wid = sid * _NC + cid
  ebase = wid * _EPW
  iota = lax.iota(jnp.int32, 16)

  @pl.loop(0, _C)
  def _zr(r):
    for j in range(_D // 16):
      vb0[r, pl.ds(16 * j, 16)] = jnp.zeros((16,), jnp.float32)

  @pl.loop(0, _STRIPE // _C)
  def _zs(t):
    pltpu.sync_copy(vb0, s_shared.at[pl.ds(sid * _STRIPE + t * _C, _C)])

  pltpu.sync_copy(vb0.at[pl.ds(0, _STRIPE % _C)],
                  s_shared.at[pl.ds(sid * _STRIPE + _STRIPE - _STRIPE % _C,
                                    _STRIPE % _C)])

  @pl.when(sid == 0)
  def _zd():
    pltpu.sync_copy(vb0, den_shared)

  pltpu.sync_copy(mx_hbm, mxv)
  plsc.subcore_barrier()
  mm = mxv[0, :]
  for r in range(1, _NW):
    mm = jnp.maximum(mm, mxv[r, :])
  for sh in (1, 2, 4, 8):
    mm = jnp.maximum(mm, mm[jnp.bitwise_xor(iota, sh)])
  mv = mm

  def istart(i, srcv, semi):
    pltpu.async_copy(src_hbm.at[pl.ds(ebase + i * _C, _C)], srcv, semi)

  def iwait(srcv, semi):
    pltpu.make_async_copy(src_hbm.at[pl.ds(ebase, _C)], srcv, semi).wait()

  def gstart(i, srcv, vb, dstv, av, semg):
    pltpu.async_copy(v_hbm.at[srcv], vb, semg)
    pltpu.async_copy(dst_hbm.at[pl.ds(ebase + i * _C, _C)], dstv, semg)
    pltpu.async_copy(a_hbm.at[pl.ds(ebase + i * _C, _C)], av, semg)

  def gwait(srcv, vb, dstv, av, semg):
    pltpu.make_async_copy(v_hbm.at[srcv], vb, semg).wait()
    pltpu.make_async_copy(dst_hbm.at[pl.ds(ebase, _C)], dstv, semg).wait()
    pltpu.make_async_copy(a_hbm.at[pl.ds(ebase, _C)], av, semg).wait()

  def compute(vb, oh, dstv, av, didx):
    @pl.loop(0, _C // 16)
    def _grp(g):
      sl = pl.ds(g * 16, 16)
      dv = dstv[sl]
      didx[sl] = lax.shift_right_logical(dv, 7)
      ev = jnp.exp(av[sl] - mv)
      for l in range(16):
        e = g * 16 + l
        sel = jnp.full((16,), l, jnp.int32)
        sc = ev[sel]
        lane = jnp.bitwise_and(dv[sel], 127)
        for j in range(8):
          vb[e, pl.ds(16 * j, 16)] = vb[e, pl.ds(16 * j, 16)] * sc
          oh[e, pl.ds(16 * j, 16)] = jnp.where(lane == j * 16 + iota,
                                               sc, 0.0)

  def sstart(vb, oh, dstv, didx, semv, semo):
    pltpu.async_copy(vb, s_shared.at[dstv], semv, add=True)
    pltpu.async_copy(oh, den_shared.at[didx], semo, add=True)

  def svwait(vb, dstv, semv):
    pltpu.make_async_copy(vb, s_shared.at[dstv], semv).wait()

  def sowait(oh, didx, semo):
    pltpu.make_async_copy(oh, den_shared.at[didx], semo).wait()

  istart(0, srcv0, semi0)
  istart(1, srcv1, semi1)
  iwait(srcv0, semi0)
  gstart(0, srcv0, vb0, dstv0, av0, semg0)
  iwait(srcv1, semi1)
  gstart(1, srcv1, vb1, dstv1, av1, semg1)

  @pl.loop(0, _NCHUNK - 1, step=2)
  def _body(i):
    gwait(srcv0, vb0, dstv0, av0, semg0)
    istart(i + 2, srcv0, semi0)

    @pl.when(i >= 1)
    def _():
      sowait(oh, didx1, semo)

    compute(vb0, oh, dstv0, av0, didx0)
    sstart(vb0, oh, dstv0, didx0, semv0, semo)

    gwait(srcv1, vb1, dstv1, av1, semg1)

    @pl.when(i + 3 < _NCHUNK)
    def _():
      istart(i + 3, srcv1, semi1)

    sowait(oh, didx0, semo)
    compute(vb1, oh, dstv1, av1, didx1)
    sstart(vb1, oh, dstv1, didx1, semv1, semo)

    svwait(vb0, dstv0, semv0)
    iwait(srcv0, semi0)
    gstart(i + 2, srcv0, vb0, dstv0, av0, semg0)

    svwait(vb1, dstv1, semv1)

    @pl.when(i + 3 < _NCHUNK)
    def _():
      iwait(srcv1, semi1)
      gstart(i + 3, srcv1, vb1, dstv1, av1, semg1)

  gwait(srcv0, vb0, dstv0, av0, semg0)
  sowait(oh, didx1, semo)
  compute(vb0, oh, dstv0, av0, didx0)
  sstart(vb0, oh, dstv0, didx0, semv0, semo)
  svwait(vb0, dstv0, semv0)
  sowait(oh, didx0, semo)

  plsc.subcore_barrier()
  pltpu.sync_copy(s_shared.at[pl.ds(sid * _STRIPE, _STRIPE)],
                  s_out.at[cid, pl.ds(sid * _STRIPE, _STRIPE)])

  @pl.when(sid == 0)
  def _od():
    pltpu.sync_copy(den_shared, den_out.at[cid])


@jax.jit
def _pass2(v, a, src, dst, mx):
  f = pl.kernel(
      _pass2_body,
      out_type=(jax.ShapeDtypeStruct((_NC, _NP, _D), jnp.float32),
                jax.ShapeDtypeStruct((_NC, _NDR, _D), jnp.float32)),
      mesh=plsc.VectorSubcoreMesh(**_mesh),
      scratch_types=(
          pltpu.VMEM((_C,), jnp.int32),
          pltpu.VMEM((_C,), jnp.int32),
          pltpu.VMEM((_C,), jnp.int32),
          pltpu.VMEM((_C,), jnp.float32),
          pltpu.VMEM((_C,), jnp.int32),
          pltpu.VMEM((_C,), jnp.int32),
          pltpu.VMEM((_C,), jnp.float32),
          pltpu.VMEM((_C,), jnp.int32),
          pltpu.VMEM((_C, _D), jnp.float32),
          pltpu.VMEM((_C, _D), jnp.float32),
          pltpu.VMEM((_C, _D), jnp.float32),
          pltpu.VMEM((_NW, 16), jnp.float32),
          pltpu.VMEM_SHARED((_NP, _D), jnp.float32),
          pltpu.VMEM_SHARED((_NDR, _D), jnp.float32),
          pltpu.SemaphoreType.DMA,
          pltpu.SemaphoreType.DMA,
          pltpu.SemaphoreType.DMA,
          pltpu.SemaphoreType.DMA,
          pltpu.SemaphoreType.DMA,
          pltpu.SemaphoreType.DMA,
          pltpu.SemaphoreType.DMA,
      ),
  )
  return f(v, a, src, dst, mx)


_BN = 1264


def _qkv_body(c_ref, x_ref, wq_ref, bq_ref, wk_ref, bk_ref, wv_ref, bv_ref,
              q_ref, k_ref, v_ref):
  c = c_ref[0, 0]
  sq = jnp.sqrt(c)
  x = x_ref[...]
  n2 = jnp.sum(x * x, axis=1, keepdims=True)
  nrm = jnp.sqrt(n2)
  z = sq * nrm
  f = (1.0 / sq) * jnp.log((1.0 + z) / (1.0 - z)) / nrm
  t = x * f
  q_ref[...] = jnp.dot(t, wq_ref[...], preferred_element_type=jnp.float32) + bq_ref[...]
  k_ref[...] = jnp.dot(t, wk_ref[...], preferred_element_type=jnp.float32) + bk_ref[...]
  v_ref[...] = jnp.dot(t, wv_ref[...], preferred_element_type=jnp.float32) + bv_ref[...]


@jax.jit
def _qkv(c2, x, wqT, bq, wkT, bk, wvT, bv):
  full = lambda: pl.BlockSpec((128, 128), lambda i: (0, 0))
  brow = lambda: pl.BlockSpec((1, 128), lambda i: (0, 0))
  blk = lambda: pl.BlockSpec((_BN, 128), lambda i: (i, 0))
  return pl.pallas_call(
      _qkv_body,
      grid=(_NP // _BN,),
      in_specs=[pl.BlockSpec((1, 1), lambda i: (0, 0)),
                blk(), full(), brow(), full(), brow(), full(), brow()],
      out_specs=[blk(), blk(), blk()],
      out_shape=[jax.ShapeDtypeStruct((_NP, _D), jnp.float32)] * 3,
  )(c2, x, wqT, bq, wkT, bk, wvT, bv)


def _fin_body(c_ref, s_ref, d_ref, o_ref):
  c = c_ref[0, 0]
  sq = jnp.sqrt(c)
  s = s_ref[...]
  t = s[0] + s[1]
  dd = d_ref[...]
  den = dd[:, 0:1] + dd[:, 1:2]
  h = t / jnp.maximum(den, 1e-16)
  n2 = jnp.sum(h * h, axis=1, keepdims=True)
  nrm = jnp.sqrt(n2)
  o_ref[...] = jnp.tanh(sq * nrm * 0.5) * h / (sq * nrm)


@jax.jit
def _finalize(c2, s, den_t):
  return pl.pallas_call(
      _fin_body,
      grid=(_NP // _BN,),
      in_specs=[pl.BlockSpec((1, 1), lambda i: (0, 0)),
                pl.BlockSpec((_NC, _BN, _D), lambda i: (0, i, 0)),
                pl.BlockSpec((_BN, _NC), lambda i: (i, 0))],
      out_specs=pl.BlockSpec((_BN, 128), lambda i: (i, 0)),
      out_shape=jax.ShapeDtypeStruct((_NP, _D), jnp.float32),
  )(c2, s, den_t)


def _mid_body(c_ref, s_ref, d_ref, wq_ref, bq_ref, wk_ref, bk_ref,
              wv_ref, bv_ref, q_ref, k_ref, v_ref):
  c = c_ref[0, 0]
  sq = jnp.sqrt(c)
  s = s_ref[...]
  t = s[0] + s[1]
  dd = d_ref[...]
  den = dd[:, 0:1] + dd[:, 1:2]
  h = t / jnp.maximum(den, 1e-16)
  n2 = jnp.sum(h * h, axis=1, keepdims=True)
  nrm = jnp.sqrt(n2)
  x = jnp.tanh(sq * nrm * 0.5) * h / (sq * nrm)
  xn2 = jnp.sum(x * x, axis=1, keepdims=True)
  xn = jnp.sqrt(xn2)
  z = sq * xn
  f = (1.0 / sq) * jnp.log((1.0 + z) / (1.0 - z)) / xn
  tt = x * f
  q_ref[...] = jnp.dot(tt, wq_ref[...], preferred_element_type=jnp.float32) + bq_ref[...]
  k_ref[...] = jnp.dot(tt, wk_ref[...], preferred_element_type=jnp.float32) + bk_ref[...]
  v_ref[...] = jnp.dot(tt, wv_ref[...], preferred_element_type=jnp.float32) + bv_ref[...]


@jax.jit
def _mid(c2, s, den_t, wqT, bq, wkT, bk, wvT, bv):
  full = lambda: pl.BlockSpec((128, 128), lambda i: (0, 0))
  brow = lambda: pl.BlockSpec((1, 128), lambda i: (0, 0))
  blk = lambda: pl.BlockSpec((_BN, 128), lambda i: (i, 0))
  return pl.pallas_call(
      _mid_body,
      grid=(_NP // _BN,),
      in_specs=[pl.BlockSpec((1, 1), lambda i: (0, 0)),
                pl.BlockSpec((_NC, _BN, _D), lambda i: (0, i, 0)),
                pl.BlockSpec((_BN, _NC), lambda i: (i, 0)),
                full(), brow(), full(), brow(), full(), brow()],
      out_specs=[blk(), blk(), blk()],
      out_shape=[jax.ShapeDtypeStruct((_NP, _D), jnp.float32)] * 3,
  )(c2, s, den_t, wqT, bq, wkT, bk, wvT, bv)


def kernel(node_embeddings, Wq0, bq0, Wk0, bk0, Wv0, bv0,
           Wq1, bq1, Wk1, bk1, Wv1, bv1, curvature, edge_index):
  c2 = curvature.reshape(1, 1).astype(jnp.float32)
  src = edge_index[0].astype(jnp.int32)
  dst = edge_index[1].astype(jnp.int32)
  x = jnp.concatenate(
      [node_embeddings.astype(jnp.float32),
       jnp.zeros((_NP - _N, _D), jnp.float32)], axis=0)
  q, k, v = _qkv(c2, x, Wq0.T, bq0.reshape(1, _D), Wk0.T, bk0.reshape(1, _D),
                 Wv0.T, bv0.reshape(1, _D))
  a, mx = _pass1(k, q, src, dst)
  s, den = _pass2(v, a, src, dst, mx)
  den_t = den.reshape(_NC, _NDR * _D)[:, :_NP].T
  q, k, v = _mid(c2, s, den_t, Wq1.T, bq1.reshape(1, _D),
                 Wk1.T, bk1.reshape(1, _D), Wv1.T, bv1.reshape(1, _D))
  a, mx = _pass1(k, q, src, dst)
  s, den = _pass2(v, a, src, dst, mx)
  den_t = den.reshape(_NC, _NDR * _D)[:, :_NP].T
  return _finalize(c2, s, den_t)[:_N]


# speedup vs baseline: 1.2019x; 1.2019x over previous
import math

import jax
import jax.numpy as jnp
from jax import lax
from jax.experimental import pallas as pl
from jax.experimental.pallas import tpu as pltpu
from jax.experimental.pallas import tpu_sc as plsc

_N = 10000
_E = 320000
_D = 128
_NP = 10240
_NDR = 80
_NC = 2
_NS = 16
_NW = _NC * _NS
_EPW = _E // _NW
_C = 80
_NCHUNK = _EPW // _C
_C1 = 128
_NCH1 = _EPW // _C1
_TAIL1 = _EPW - _NCH1 * _C1
_STRIPE = _NP // _NS
_INV_SCALE = 1.0 / math.sqrt(_D)

_mesh = dict(core_axis_name="c", subcore_axis_name="s", num_cores=_NC,
             num_subcores=_NS)


def _pass1_body(k_hbm, q_hbm, src_hbm, dst_hbm, a_out, mx_out,
                srcall, dstall, kr0, qr0, kr1, qr1, abuf, mbuf,
                sem0, sem1):
  cid = lax.axis_index("c")
  sid = lax.axis_index("s")
  wid = sid * _NC + cid
  ebase = wid * _EPW
  mbuf[...] = jnp.full((16,), -jnp.inf, jnp.float32)
  pltpu.sync_copy(src_hbm.at[pl.ds(ebase, _EPW)], srcall)
  pltpu.sync_copy(dst_hbm.at[pl.ds(ebase, _EPW)], dstall)
  iota = lax.iota(jnp.int32, 16)

  def start(i, kr, qr, sem, n=_C1):
    pltpu.async_copy(k_hbm.at[srcall.at[pl.ds(i * _C1, n)]],
                     kr.at[pl.ds(0, n)], sem)
    pltpu.async_copy(q_hbm.at[dstall.at[pl.ds(i * _C1, n)]],
                     qr.at[pl.ds(0, n)], sem)

  def wait(kr, qr, sem, n=_C1):
    pltpu.make_async_copy(k_hbm.at[srcall.at[pl.ds(0, n)]],
                          kr.at[pl.ds(0, n)], sem).wait()
    pltpu.make_async_copy(q_hbm.at[dstall.at[pl.ds(0, n)]],
                          qr.at[pl.ds(0, n)], sem).wait()

  def compute(i, kr, qr, n=_C1):
    @pl.loop(0, n // 16)
    def _grp(g):
      res = jnp.zeros((16,), jnp.float32)
      for l in range(16):
        e = g * 16 + l
        acc = kr[e, pl.ds(0, 16)] * qr[e, pl.ds(0, 16)]
        for j in range(1, 8):
          acc = acc + kr[e, pl.ds(16 * j, 16)] * qr[e, pl.ds(16 * j, 16)]
        for sh in (1, 2, 4, 8):
          acc = acc + acc[jnp.bitwise_xor(iota, sh)]
        res = jnp.where(iota == l, acc, res)
      abuf[pl.ds(i * _C1 + g * 16, 16)] = res * _INV_SCALE

  start(0, kr0, qr0, sem0)

  @pl.loop(0, _NCH1 - 1, step=2)
  def _body(i):
    start(i + 1, kr1, qr1, sem1)
    wait(kr0, qr0, sem0)
    compute(i, kr0, qr0)

    @pl.when(i + 2 < _NCH1)
    def _():
      start(i + 2, kr0, qr0, sem0)

    wait(kr1, qr1, sem1)
    compute(i + 1, kr1, qr1)

  start(_NCH1, kr0, qr0, sem0, n=_TAIL1)
  wait(kr0, qr0, sem0, n=_TAIL1)
  compute(_NCH1, kr0, qr0, n=_TAIL1)

  @pl.loop(0, _EPW // 16)
  def _mx(t):
    mbuf[...] = jnp.maximum(mbuf[...], abuf[pl.ds(t * 16, 16)])

  pltpu.sync_copy(abuf, a_out.at[pl.ds(ebase, _EPW)])
  pltpu.sync_copy(mbuf, mx_out.at[wid])


@jax.jit
def _pass1(k, q, src, dst):
  f = pl.kernel(
      _pass1_body,
      out_type=(jax.ShapeDtypeStruct((_E,), jnp.float32),
                jax.ShapeDtypeStruct((_NW, 16), jnp.float32)),
      mesh=plsc.VectorSubcoreMesh(**_mesh),
      scratch_types=(
          pltpu.VMEM((_EPW,), jnp.int32),
          pltpu.VMEM((_EPW,), jnp.int32),
          pltpu.VMEM((_C1, _D), jnp.float32),
          pltpu.VMEM((_C1, _D), jnp.float32),
          pltpu.VMEM((_C1, _D), jnp.float32),
          pltpu.VMEM((_C1, _D), jnp.float32),
          pltpu.VMEM((_EPW,), jnp.float32),
          pltpu.VMEM((16,), jnp.float32),
          pltpu.SemaphoreType.DMA,
          pltpu.SemaphoreType.DMA,
      ),
  )
  return f(k, q, src, dst)


def _pass2_body(v_hbm, a_hbm, src_hbm, dst_hbm, mx_hbm, s_out, den_out,
                srcall, dstv0, av0, dstv1, av1, didx, vb0, vb1, oh,
                mxv, s_shared, den_shared, sem0, sem1):
  cid = lax.axis_index("c")
  sid = lax.axis_index("s")
  wid = sid * _NC + cid
  ebase = wid * _EPW
  iota = lax.iota(jnp.int32, 16)

  @pl.loop(0, _C)
  def _zr(r):
    for j in range(_D // 16):
      vb0[r, pl.ds(16 * j, 16)] = jnp.zeros((16,), jnp.float32)

  @pl.loop(0, _STRIPE // _C)
  def _zs(t):
    pltpu.sync_copy(vb0, s_shared.at[pl.ds(sid * _STRIPE + t * _C, _C)])

  @pl.when(sid == 0)
  def _zd():
    pltpu.sync_copy(vb0, den_shared)

  pltpu.sync_copy(src_hbm.at[pl.ds(ebase, _EPW)], srcall)
  pltpu.sync_copy(mx_hbm, mxv)
  plsc.subcore_barrier()
  mm = mxv[0, :]
  for r in range(1, _NW):
    mm = jnp.maximum(mm, mxv[r, :])
  for sh in (1, 2, 4, 8):
    mm = jnp.maximum(mm, mm[jnp.bitwise_xor(iota, sh)])
  mv = mm

  def start(i, vb, dstv, av, sem):
    pltpu.async_copy(v_hbm.at[srcall.at[pl.ds(i * _C, _C)]], vb, sem)
    pltpu.async_copy(dst_hbm.at[pl.ds(ebase + i * _C, _C)], dstv, sem)
    pltpu.async_copy(a_hbm.at[pl.ds(ebase + i * _C, _C)], av, sem)

  def wait(vb, dstv, av, sem):
    pltpu.make_async_copy(v_hbm.at[srcall.at[pl.ds(0, _C)]], vb, sem).wait()
    pltpu.make_async_copy(dst_hbm.at[pl.ds(ebase, _C)], dstv, sem).wait()
    pltpu.make_async_copy(a_hbm.at[pl.ds(ebase, _C)], av, sem).wait()

  def compute_scatter(vb, dstv, av):
    for g in range(_C // 16):
      sl = pl.ds(g * 16, 16)
      didx[sl] = lax.shift_right_logical(dstv[sl], 7)

    @pl.loop(0, _C // 16)
    def _grp(g):
      ev = jnp.exp(av[pl.ds(g * 16, 16)] - mv)
      for l in range(16):
        e = g * 16 + l
        sc = ev[jnp.full((16,), l, jnp.int32)]
        for j in range(8):
          vb[e, pl.ds(16 * j, 16)] = vb[e, pl.ds(16 * j, 16)] * sc

    pltpu.sync_copy(vb, s_shared.at[dstv], add=True)

    @pl.loop(0, _C // 16)
    def _grp2(g):
      ev = jnp.exp(av[pl.ds(g * 16, 16)] - mv)
      dv = dstv[pl.ds(g * 16, 16)]
      for l in range(16):
        e = g * 16 + l
        sel = jnp.full((16,), l, jnp.int32)
        sc = ev[sel]
        lane = jnp.bitwise_and(dv[sel], 127)
        for j in range(8):
          oh[e, pl.ds(16 * j, 16)] = jnp.where(lane == j * 16 + iota,
                                               sc, 0.0)

    pltpu.sync_copy(oh, den_shared.at[didx], add=True)

  start(0, vb0, dstv0, av0, sem0)

  @pl.loop(0, _NCHUNK - 1, step=2)
  def _body(i):
    start(i + 1, vb1, dstv1, av1, sem1)
    wait(vb0, dstv0, av0, sem0)
    compute_scatter(vb0, dstv0, av0)
    start(i + 2, vb0, dstv0, av0, sem0)
    wait(vb1, dstv1, av1, sem1)
    compute_scatter(vb1, dstv1, av1)

  wait(vb0, dstv0, av0, sem0)
  compute_scatter(vb0, dstv0, av0)

  plsc.subcore_barrier()
  pltpu.sync_copy(s_shared.at[pl.ds(sid * _STRIPE, _STRIPE)],
                  s_out.at[cid, pl.ds(sid * _STRIPE, _STRIPE)])

  @pl.when(sid == 0)
  def _od():
    pltpu.sync_copy(den_shared, den_out.at[cid])


@jax.jit
def _pass2(v, a, src, dst, mx):
  f = pl.kernel(
      _pass2_body,
      out_type=(jax.ShapeDtypeStruct((_NC, _NP, _D), jnp.float32),
                jax.ShapeDtypeStruct((_NC, _NDR, _D), jnp.float32)),
      mesh=plsc.VectorSubcoreMesh(**_mesh),
      scratch_types=(
          pltpu.VMEM((_EPW,), jnp.int32),
          pltpu.VMEM((_C,), jnp.int32),
          pltpu.VMEM((_C,), jnp.float32),
          pltpu.VMEM((_C,), jnp.int32),
          pltpu.VMEM((_C,), jnp.float32),
          pltpu.VMEM((_C,), jnp.int32),
          pltpu.VMEM((_C, _D), jnp.float32),
          pltpu.VMEM((_C, _D), jnp.float32),
          pltpu.VMEM((_C, _D), jnp.float32),
          pltpu.VMEM((_NW, 16), jnp.float32),
          pltpu.VMEM_SHARED((_NP, _D), jnp.float32),
          pltpu.VMEM_SHARED((_NDR, _D), jnp.float32),
          pltpu.SemaphoreType.DMA,
          pltpu.SemaphoreType.DMA,
      ),
  )
  return f(v, a, src, dst, mx)


_BN = 1280


def _qkv_body(c_ref, x_ref, wq_ref, bq_ref, wk_ref, bk_ref, wv_ref, bv_ref,
              q_ref, k_ref, v_ref):
  c = c_ref[0, 0]
  sq = jnp.sqrt(c)
  x = x_ref[...]
  n2 = jnp.sum(x * x, axis=1, keepdims=True)
  nrm = jnp.sqrt(n2)
  z = sq * nrm
  f = (1.0 / sq) * jnp.log((1.0 + z) / (1.0 - z)) / nrm
  t = x * f
  q_ref[...] = jnp.dot(t, wq_ref[...], preferred_element_type=jnp.float32) + bq_ref[...]
  k_ref[...] = jnp.dot(t, wk_ref[...], preferred_element_type=jnp.float32) + bk_ref[...]
  v_ref[...] = jnp.dot(t, wv_ref[...], preferred_element_type=jnp.float32) + bv_ref[...]


@jax.jit
def _qkv(c2, x, wqT, bq, wkT, bk, wvT, bv):
  full = lambda: pl.BlockSpec((128, 128), lambda i: (0, 0))
  brow = lambda: pl.BlockSpec((1, 128), lambda i: (0, 0))
  blk = lambda: pl.BlockSpec((_BN, 128), lambda i: (i, 0))
  return pl.pallas_call(
      _qkv_body,
      grid=(_NP // _BN,),
      in_specs=[pl.BlockSpec((1, 1), lambda i: (0, 0)),
                blk(), full(), brow(), full(), brow(), full(), brow()],
      out_specs=[blk(), blk(), blk()],
      out_shape=[jax.ShapeDtypeStruct((_NP, _D), jnp.float32)] * 3,
  )(c2, x, wqT, bq, wkT, bk, wvT, bv)


def _fin_body(c_ref, s_ref, d_ref, o_ref):
  c = c_ref[0, 0]
  sq = jnp.sqrt(c)
  s = s_ref[...]
  t = s[0] + s[1]
  dd = d_ref[...]
  den = dd[:, 0:1] + dd[:, 1:2]
  h = t / jnp.maximum(den, 1e-16)
  n2 = jnp.sum(h * h, axis=1, keepdims=True)
  nrm = jnp.sqrt(n2)
  o_ref[...] = jnp.tanh(sq * nrm * 0.5) * h / (sq * nrm)


@jax.jit
def _finalize(c2, s, den_t):
  return pl.pallas_call(
      _fin_body,
      grid=(_NP // _BN,),
      in_specs=[pl.BlockSpec((1, 1), lambda i: (0, 0)),
                pl.BlockSpec((_NC, _BN, _D), lambda i: (0, i, 0)),
                pl.BlockSpec((_BN, _NC), lambda i: (i, 0))],
      out_specs=pl.BlockSpec((_BN, 128), lambda i: (i, 0)),
      out_shape=jax.ShapeDtypeStruct((_NP, _D), jnp.float32),
  )(c2, s, den_t)


def _mid_body(c_ref, s_ref, d_ref, wq_ref, bq_ref, wk_ref, bk_ref,
              wv_ref, bv_ref, q_ref, k_ref, v_ref):
  c = c_ref[0, 0]
  sq = jnp.sqrt(c)
  s = s_ref[...]
  t = s[0] + s[1]
  dd = d_ref[...]
  den = dd[:, 0:1] + dd[:, 1:2]
  h = t / jnp.maximum(den, 1e-16)
  n2 = jnp.sum(h * h, axis=1, keepdims=True)
  nrm = jnp.sqrt(n2)
  x = jnp.tanh(sq * nrm * 0.5) * h / (sq * nrm)
  xn2 = jnp.sum(x * x, axis=1, keepdims=True)
  xn = jnp.sqrt(xn2)
  z = sq * xn
  f = (1.0 / sq) * jnp.log((1.0 + z) / (1.0 - z)) / xn
  tt = x * f
  q_ref[...] = jnp.dot(tt, wq_ref[...], preferred_element_type=jnp.float32) + bq_ref[...]
  k_ref[...] = jnp.dot(tt, wk_ref[...], preferred_element_type=jnp.float32) + bk_ref[...]
  v_ref[...] = jnp.dot(tt, wv_ref[...], preferred_element_type=jnp.float32) + bv_ref[...]


@jax.jit
def _mid(c2, s, den_t, wqT, bq, wkT, bk, wvT, bv):
  full = lambda: pl.BlockSpec((128, 128), lambda i: (0, 0))
  brow = lambda: pl.BlockSpec((1, 128), lambda i: (0, 0))
  blk = lambda: pl.BlockSpec((_BN, 128), lambda i: (i, 0))
  return pl.pallas_call(
      _mid_body,
      grid=(_NP // _BN,),
      in_specs=[pl.BlockSpec((1, 1), lambda i: (0, 0)),
                pl.BlockSpec((_NC, _BN, _D), lambda i: (0, i, 0)),
                pl.BlockSpec((_BN, _NC), lambda i: (i, 0)),
                full(), brow(), full(), brow(), full(), brow()],
      out_specs=[blk(), blk(), blk()],
      out_shape=[jax.ShapeDtypeStruct((_NP, _D), jnp.float32)] * 3,
  )(c2, s, den_t, wqT, bq, wkT, bk, wvT, bv)


def kernel(node_embeddings, Wq0, bq0, Wk0, bk0, Wv0, bv0,
           Wq1, bq1, Wk1, bk1, Wv1, bv1, curvature, edge_index):
  c2 = curvature.reshape(1, 1).astype(jnp.float32)
  src = edge_index[0].astype(jnp.int32)
  dst = edge_index[1].astype(jnp.int32)
  x = jnp.concatenate(
      [node_embeddings.astype(jnp.float32),
       jnp.zeros((_NP - _N, _D), jnp.float32)], axis=0)
  q, k, v = _qkv(c2, x, Wq0.T, bq0.reshape(1, _D), Wk0.T, bk0.reshape(1, _D),
                 Wv0.T, bv0.reshape(1, _D))
  a, mx = _pass1(k, q, src, dst)
  s, den = _pass2(v, a, src, dst, mx)
  den_t = den.reshape(_NC, _NDR * _D)[:, :_NP].T
  q, k, v = _mid(c2, s, den_t, Wq1.T, bq1.reshape(1, _D),
                 Wk1.T, bk1.reshape(1, _D), Wv1.T, bv1.reshape(1, _D))
  a, mx = _pass1(k, q, src, dst)
  s, den = _pass2(v, a, src, dst, mx)
  den_t = den.reshape(_NC, _NDR * _D)[:, :_NP].T
  return _finalize(c2, s, den_t)[:_N]
